# Initial kernel scaffold; baseline (speedup 1.0000x reference)
import jax
import jax.numpy as jnp
from jax.experimental import pallas as pl


def kernel(heads, q_embeddings, q_word_h, attention_mask, entity_emb, W_match, b_match, W_steps, b_steps, W_rel, b_rel, W_hop, b_hop, relation_importance, temperature, triples_subj, triples_rel, triples_obj):
    raise NotImplementedError("write your pallas kernel here")


# speedup vs baseline: 7.0435x; 7.0435x over previous
import math

import jax
import jax.numpy as jnp
from jax import lax
from jax.experimental import pallas as pl
from jax.experimental.pallas import tpu as pltpu
from jax.experimental.pallas import tpu_sc as plsc

NUM_WAYS = 2
NUM_STEPS = 3
N_TILES = 16
CHUNK = 512
SUBG = 128
NSUBG = CHUNK // SUBG
SUBS = 64
NSUBS = CHUNK // SUBS
NSLOT = 4


def _relmix_body(q_ref, qwh_ref, mask_ref, Wm_ref, bm_ref, Wst_ref, bst_ref,
                 Wrel_ref, brel_ref, imp_ref, temp_ref, qmT_ref, wr16_ref):
    q = q_ref[...]
    qwh = qwh_ref[...]
    mask = mask_ref[...]
    temp = temp_ref[0, 0]
    qmT_ref[...] = (q @ Wm_ref[...] + bm_ref[...]).T
    W = Wst_ref[0, 0]
    b = bst_ref[0, 0]
    cq = jnp.tanh(q @ W + b)
    ql = jnp.sum(cq[:, None, :] * qwh, axis=2) / temp
    qd = jax.nn.softmax(ql, axis=1) * mask
    qd = qd / (jnp.sum(qd, axis=1, keepdims=True) + 1e-6)
    ctx = jnp.sum(qd[:, :, None] * qwh, axis=1)
    rl = ctx @ Wrel_ref[0] + brel_ref[0]
    rd = jax.nn.softmax(rl, axis=1)
    wr = rd * imp_ref[...]
    wrt = wr.T
    wr16_ref[0, 0] = jnp.concatenate([wrt, wrt], axis=1)


def _direct_body(emb_ref, qmT_ref, headsT_ref, enh16_ref, sums_ref):
    k = pl.program_id(0)
    d = emb_ref.shape[1]
    z = (emb_ref[...] @ qmT_ref[...]) * (1.0 / math.sqrt(d))
    dt = jax.nn.sigmoid(z)
    enh = headsT_ref[...] * (1.0 + 0.3 * dt)
    e16 = jnp.concatenate([enh, enh], axis=1)
    enh16_ref[...] = e16

    @pl.when(k == 0)
    def _():
        sums_ref[...] = jnp.zeros_like(sums_ref)

    sums_ref[...] += jnp.sum(e16, axis=0, keepdims=True)


def _combine_body(probs_ref, q_ref, Whop_ref, bhop_ref, outT_ref):
    q = q_ref[...]
    h0 = jax.nn.softmax(q @ Whop_ref[0] + bhop_ref[0], axis=1)
    h1 = jax.nn.softmax(q @ Whop_ref[1] + bhop_ref[1], axis=1)
    wgt = 0.5 * jnp.concatenate([h0.T, h1.T], axis=0)
    p8 = probs_ref[...][:, :, :8]
    outT_ref[...] = jnp.sum(p8 * wgt[:, None, :], axis=0)


def _make_sc_kernel(E, T, R):
    rp = E // N_TILES
    tpt = T // N_TILES
    nch = tpt // CHUNK

    def body(enh16_h, sums_h, wr3_h, subj_h, rel3_h, obj2_h, eall_h,
             acc_tab, wr_tab, mx_tab, idx_s, idx_o, idx_r, egbuf,
             rgbuf, nbuf, stage8, sumbuf, mxall):
        c = lax.axis_index("c")
        s = lax.axis_index("s")
        row0 = pl.multiple_of(s * rp, 8)
        tri0 = pl.multiple_of(s * tpt, 8)
        zval = jnp.zeros((16,), jnp.float32)

        @pl.when(s == 0)
        def _():
            pltpu.sync_copy(wr3_h.at[c], wr_tab)

        pltpu.sync_copy(sums_h, sumbuf)
        hs = sumbuf[0, :]
        hs = jnp.where(hs > 0.0, hs, 1.0)
        hinv = 1.0 / hs
        pltpu.sync_copy(enh16_h.at[pl.ds(row0, rp)], nbuf)

        def scale0(j, _):
            nbuf[j, :] = nbuf[j, :] * hinv
            return 0

        lax.fori_loop(0, rp, scale0, 0)
        e0_off = pl.multiple_of(c * NSLOT * E + row0, 8)
        pltpu.sync_copy(nbuf, eall_h.at[pl.ds(e0_off, rp)])

        def zero(j, _):
            nbuf[j, :] = zval
            return 0

        lax.fori_loop(0, rp, zero, 0)
        pltpu.sync_copy(nbuf, acc_tab.at[pl.ds(row0, rp)])
        plsc.subcore_barrier()

        nvec = CHUNK // 16
        for t in range(NUM_STEPS):

            def chunk(g, _):
                t0 = pl.multiple_of(tri0 + g * CHUNK, 8)
                pltpu.sync_copy(subj_h.at[pl.ds(t0, CHUNK)], idx_s)
                pltpu.sync_copy(rel3_h.at[pl.ds(t * T + t0, CHUNK)], idx_r)
                pltpu.sync_copy(obj2_h.at[pl.ds(pl.multiple_of(t0 // SUBS, 8),
                                                NSUBS)], idx_o)
                src_base = (c * NSLOT + t) * E

                def shift(i, _):
                    sl = pl.ds(pl.multiple_of(i * 16, 8), 16)
                    idx_s[sl] = idx_s[sl] + src_base
                    return 0

                lax.fori_loop(0, nvec, shift, 0)
                for j in range(NSUBG):
                    pltpu.sync_copy(eall_h.at[idx_s.at[pl.ds(j * SUBG, SUBG)]],
                                    egbuf.at[pl.ds(j * SUBG, SUBG)])
                    pltpu.sync_copy(wr_tab.at[idx_r.at[pl.ds(j * SUBG, SUBG)]],
                                    rgbuf.at[pl.ds(j * SUBG, SUBG)])

                def mul(m, _):
                    egbuf[m, :] = egbuf[m, :] * rgbuf[m, :]
                    return 0

                lax.fori_loop(0, CHUNK, mul, 0)
                for j in range(NSUBS):
                    pltpu.sync_copy(egbuf.at[pl.ds(j * SUBS, SUBS)],
                                    acc_tab.at[idx_o.at[j]], add=True)
                return 0

            lax.fori_loop(0, nch, chunk, 0)
            plsc.subcore_barrier()

            pltpu.sync_copy(acc_tab.at[pl.ds(row0, rp)], nbuf)

            def clipmax(j, mx):
                v = nbuf[j, :]
                v = jnp.minimum(jnp.maximum(v, 0.0), 1.0)
                nbuf[j, :] = v
                return jnp.maximum(mx, v)

            mx = lax.fori_loop(0, rp, clipmax, zval)
            stage8[0, :] = mx
            pltpu.sync_copy(stage8,
                            mx_tab.at[pl.ds(pl.multiple_of(s * 8, 8), 8)])
            plsc.subcore_barrier()
            pltpu.sync_copy(mx_tab, mxall)

            def red(j, m):
                return jnp.maximum(m, mxall[j * 8, :])

            gmx = lax.fori_loop(0, N_TILES, red, zval)
            gmx = jnp.where(gmx > 0.0, gmx, 1.0)
            ginv = 1.0 / gmx

            def scale(j, _):
                nbuf[j, :] = nbuf[j, :] * ginv
                return 0

            lax.fori_loop(0, rp, scale, 0)
            out_off = pl.multiple_of((c * NSLOT + 1 + t) * E + row0, 8)
            pltpu.sync_copy(nbuf, eall_h.at[pl.ds(out_off, rp)])
            if t < NUM_STEPS - 1:
                lax.fori_loop(0, rp, zero, 0)
                pltpu.sync_copy(nbuf, acc_tab.at[pl.ds(row0, rp)])
            plsc.subcore_barrier()

    mesh = plsc.VectorSubcoreMesh(core_axis_name="c", subcore_axis_name="s",
                                  num_cores=NUM_WAYS, num_subcores=N_TILES)
    return pl.kernel(
        body,
        out_type=jax.ShapeDtypeStruct((NUM_WAYS * NSLOT * E, 16),
                                      jnp.float32),
        mesh=mesh,
        compiler_params=pltpu.CompilerParams(use_tc_tiling_on_sc=False),
        scratch_types=[
            pltpu.VMEM_SHARED((E, 16), jnp.float32),
            pltpu.VMEM_SHARED((NUM_STEPS * R, 16), jnp.float32),
            pltpu.VMEM_SHARED((N_TILES * 8, 16), jnp.float32),
            pltpu.VMEM((CHUNK,), jnp.int32),
            pltpu.VMEM((NSUBS, SUBS), jnp.int32),
            pltpu.VMEM((CHUNK,), jnp.int32),
            pltpu.VMEM((CHUNK, 16), jnp.float32),
            pltpu.VMEM((CHUNK, 16), jnp.float32),
            pltpu.VMEM((E // N_TILES, 16), jnp.float32),
            pltpu.VMEM((8, 16), jnp.float32),
            pltpu.VMEM((1, 16), jnp.float32),
            pltpu.VMEM((N_TILES * 8, 16), jnp.float32),
        ],
    )


def kernel(heads, q_embeddings, q_word_h, attention_mask, entity_emb,
           W_match, b_match, W_steps, b_steps, W_rel, b_rel, W_hop, b_hop,
           relation_importance, temperature,
           triples_subj, triples_rel, triples_obj):
    E, D = entity_emb.shape
    B = heads.shape[0]
    L = q_word_h.shape[1]
    R = W_rel.shape[2]
    T = triples_subj.shape[0]
    EB = 1000

    f32 = jnp.float32
    imp2 = relation_importance.reshape(1, R).astype(f32)
    temp2 = jnp.asarray(temperature, f32).reshape(1, 1)
    bm2 = b_match.reshape(1, D)

    qmT, wr16 = pl.pallas_call(
        _relmix_body,
        grid=(NUM_WAYS, NUM_STEPS),
        in_specs=[
            pl.BlockSpec((B, D), lambda w, t: (0, 0)),
            pl.BlockSpec((B, L, D), lambda w, t: (0, 0, 0)),
            pl.BlockSpec((B, L), lambda w, t: (0, 0)),
            pl.BlockSpec((D, D), lambda w, t: (0, 0)),
            pl.BlockSpec((1, D), lambda w, t: (0, 0)),
            pl.BlockSpec((1, 1, D, D), lambda w, t: (w, t, 0, 0)),
            pl.BlockSpec((1, 1, D), lambda w, t: (w * NUM_STEPS + t, 0, 0)),
            pl.BlockSpec((1, D, R), lambda w, t: (w, 0, 0)),
            pl.BlockSpec((1, 1, R), lambda w, t: (w, 0, 0)),
            pl.BlockSpec((1, R), lambda w, t: (0, 0)),
            pl.BlockSpec((1, 1), lambda w, t: (0, 0)),
        ],
        out_specs=[
            pl.BlockSpec((D, B), lambda w, t: (0, 0)),
            pl.BlockSpec((1, 1, R, 16), lambda w, t: (w, t, 0, 0)),
        ],
        out_shape=[
            jax.ShapeDtypeStruct((D, B), f32),
            jax.ShapeDtypeStruct((NUM_WAYS, NUM_STEPS, R, 16), f32),
        ],
    )(q_embeddings, q_word_h, attention_mask, W_match, bm2, W_steps,
      b_steps.reshape(NUM_WAYS * NUM_STEPS, 1, D),
      W_rel, b_rel.reshape(NUM_WAYS, 1, R), imp2, temp2)

    headsT = heads.T
    enh16, sums = pl.pallas_call(
        _direct_body,
        grid=(E // EB,),
        in_specs=[
            pl.BlockSpec((EB, D), lambda k: (k, 0)),
            pl.BlockSpec((D, B), lambda k: (0, 0)),
            pl.BlockSpec((EB, B), lambda k: (k, 0)),
        ],
        out_specs=[
            pl.BlockSpec((EB, 16), lambda k: (k, 0)),
            pl.BlockSpec((1, 16), lambda k: (0, 0)),
        ],
        out_shape=[
            jax.ShapeDtypeStruct((E, 16), f32),
            jax.ShapeDtypeStruct((1, 16), f32),
        ],
    )(entity_emb, qmT, headsT)

    EP = -(-E // (8 * N_TILES)) * (8 * N_TILES)
    tpt_p = -(-(T // N_TILES) // CHUNK) * CHUNK
    TP = tpt_p * N_TILES
    i32 = jnp.int32
    subjp = jnp.concatenate(
        [triples_subj, jnp.full((TP - T,), E, i32)])
    relp = jnp.concatenate([triples_rel, jnp.zeros((TP - T,), i32)])
    objp = jnp.concatenate([triples_obj, jnp.full((TP - T,), E, i32)])
    wr3 = wr16.reshape(NUM_WAYS, NUM_STEPS * R, 16)
    rel3 = (relp[None, :]
            + (jnp.arange(NUM_STEPS, dtype=i32) * R)[:, None]).reshape(-1)
    obj2 = objp.reshape(TP // SUBS, SUBS)
    enh16p = jnp.concatenate(
        [enh16, jnp.zeros((EP - E, 16), f32)], axis=0)
    eall = _make_sc_kernel(EP, TP, R)(enh16p, sums, wr3, subjp, rel3, obj2)

    probs3 = eall.reshape(NUM_WAYS, NSLOT, EP, 16)[:, 1:, :E].reshape(
        NUM_WAYS * NUM_STEPS, E, 16)
    outT = pl.pallas_call(
        _combine_body,
        grid=(E // EB,),
        in_specs=[
            pl.BlockSpec((NUM_WAYS * NUM_STEPS, EB, 16), lambda k: (0, k, 0)),
            pl.BlockSpec((B, D), lambda k: (0, 0)),
            pl.BlockSpec((NUM_WAYS, D, NUM_STEPS), lambda k: (0, 0, 0)),
            pl.BlockSpec((NUM_WAYS, NUM_STEPS), lambda k: (0, 0)),
        ],
        out_specs=pl.BlockSpec((EB, B), lambda k: (k, 0)),
        out_shape=jax.ShapeDtypeStruct((E, B), f32),
    )(probs3, q_embeddings, W_hop, b_hop)

    return outT.T


# speedup vs baseline: 10.9388x; 1.5530x over previous
import math

import jax
import jax.numpy as jnp
from jax import lax
from jax.experimental import pallas as pl
from jax.experimental.pallas import tpu as pltpu
from jax.experimental.pallas import tpu_sc as plsc

NUM_WAYS = 2
NUM_STEPS = 3
N_TILES = 16
CHUNK = 512
SUBG = 128
NSUBG = CHUNK // SUBG
SUBS = 64
NSUBS = CHUNK // SUBS
NSLOT = 4


def _relmix_body(q_ref, qwh_ref, mask_ref, Wm_ref, bm_ref, Wst_ref, bst_ref,
                 Wrel_ref, brel_ref, imp_ref, temp_ref, qmT_ref, wr16_ref):
    q = q_ref[...]
    qwh = qwh_ref[...]
    mask = mask_ref[...]
    temp = temp_ref[0, 0]
    qmT_ref[...] = (q @ Wm_ref[...] + bm_ref[...]).T
    W = Wst_ref[0, 0]
    b = bst_ref[0, 0]
    cq = jnp.tanh(q @ W + b)
    ql = jnp.sum(cq[:, None, :] * qwh, axis=2) / temp
    qd = jax.nn.softmax(ql, axis=1) * mask
    qd = qd / (jnp.sum(qd, axis=1, keepdims=True) + 1e-6)
    ctx = jnp.sum(qd[:, :, None] * qwh, axis=1)
    rl = ctx @ Wrel_ref[0] + brel_ref[0]
    rd = jax.nn.softmax(rl, axis=1)
    wr = rd * imp_ref[...]
    wrt = wr.T
    wr16_ref[0, 0] = jnp.concatenate([wrt, wrt], axis=1)


def _direct_body(emb_ref, qmT_ref, headsT_ref, enh16_ref, sums_ref):
    k = pl.program_id(0)
    d = emb_ref.shape[1]
    z = (emb_ref[...] @ qmT_ref[...]) * (1.0 / math.sqrt(d))
    dt = jax.nn.sigmoid(z)
    enh = headsT_ref[...] * (1.0 + 0.3 * dt)
    e16 = jnp.concatenate([enh, enh], axis=1)
    enh16_ref[...] = e16

    @pl.when(k == 0)
    def _():
        sums_ref[...] = jnp.zeros_like(sums_ref)

    sums_ref[...] += jnp.sum(e16, axis=0, keepdims=True)


def _combine_body(probs_ref, q_ref, Whop_ref, bhop_ref, outT_ref):
    q = q_ref[...]
    h0 = jax.nn.softmax(q @ Whop_ref[0] + bhop_ref[0], axis=1)
    h1 = jax.nn.softmax(q @ Whop_ref[1] + bhop_ref[1], axis=1)
    wgt = 0.5 * jnp.concatenate([h0.T, h1.T], axis=0)
    p8 = probs_ref[...][:, :, :8]
    outT_ref[...] = jnp.sum(p8 * wgt[:, None, :], axis=0)


def _make_sc_kernel(E, T, R):
    rp = E // N_TILES
    tpt = T // N_TILES
    nch = tpt // CHUNK

    def body(enh16_h, sums_h, wr3_h, subj_h, rel3_h, obj2_h, eall_h,
             acc_tab, wr_tab, mx_tab, idx_s, idx_o, idx_r, egbuf,
             rgbuf, nbuf, stage8, sumbuf, mxall, isem, gsem, ssem):
        c = lax.axis_index("c")
        s = lax.axis_index("s")
        row0 = pl.multiple_of(s * rp, 8)
        tri0 = pl.multiple_of(s * tpt, 8)
        zval = jnp.zeros((16,), jnp.float32)

        @pl.when(s == 0)
        def _():
            pltpu.sync_copy(wr3_h.at[c], wr_tab)

        pltpu.sync_copy(sums_h, sumbuf)
        hs = sumbuf[0, :]
        hs = jnp.where(hs > 0.0, hs, 1.0)
        hinv = 1.0 / hs
        pltpu.sync_copy(enh16_h.at[pl.ds(row0, rp)], nbuf)

        def scale0(j, _):
            nbuf[j, :] = nbuf[j, :] * hinv
            return 0

        lax.fori_loop(0, rp, scale0, 0)
        e0_off = pl.multiple_of(c * NSLOT * E + row0, 8)
        pltpu.sync_copy(nbuf, eall_h.at[pl.ds(e0_off, rp)])

        def zero(j, _):
            nbuf[j, :] = zval
            return 0

        lax.fori_loop(0, rp, zero, 0)
        pltpu.sync_copy(nbuf, acc_tab.at[pl.ds(row0, rp)])
        plsc.subcore_barrier()

        nvec = CHUNK // 16
        for t in range(NUM_STEPS):

            def chunk(g, _):
                t0 = pl.multiple_of(tri0 + g * CHUNK, 8)
                di = [
                    pltpu.async_copy(subj_h.at[pl.ds(t0, CHUNK)], idx_s,
                                     isem),
                    pltpu.async_copy(rel3_h.at[pl.ds(t * T + t0, CHUNK)],
                                     idx_r, isem),
                    pltpu.async_copy(
                        obj2_h.at[pl.ds(pl.multiple_of(t0 // SUBS, 8),
                                        NSUBS)], idx_o, isem),
                ]
                for d in di:
                    d.wait()
                src_base = (c * NSLOT + t) * E

                def shift(i, _):
                    sl = pl.ds(pl.multiple_of(i * 16, 8), 16)
                    idx_s[sl] = idx_s[sl] + src_base
                    return 0

                lax.fori_loop(0, nvec, shift, 0)
                dg = []
                for j in range(NSUBG):
                    dg.append(pltpu.async_copy(
                        eall_h.at[idx_s.at[pl.ds(j * SUBG, SUBG)]],
                        egbuf.at[pl.ds(j * SUBG, SUBG)], gsem))
                for j in range(NSUBG):
                    pltpu.sync_copy(wr_tab.at[idx_r.at[pl.ds(j * SUBG, SUBG)]],
                                    rgbuf.at[pl.ds(j * SUBG, SUBG)])
                for d in dg:
                    d.wait()

                def mul(m, _):
                    egbuf[m, :] = egbuf[m, :] * rgbuf[m, :]
                    return 0

                lax.fori_loop(0, CHUNK, mul, 0)
                ds_ = []
                for j in range(NSUBS):
                    ds_.append(pltpu.async_copy(
                        egbuf.at[pl.ds(j * SUBS, SUBS)],
                        acc_tab.at[idx_o.at[j]], ssem, add=True))
                for d in ds_:
                    d.wait()
                return 0

            lax.fori_loop(0, nch, chunk, 0)
            plsc.subcore_barrier()

            pltpu.sync_copy(acc_tab.at[pl.ds(row0, rp)], nbuf)

            def clipmax(j, mx):
                v = nbuf[j, :]
                v = jnp.minimum(jnp.maximum(v, 0.0), 1.0)
                nbuf[j, :] = v
                return jnp.maximum(mx, v)

            mx = lax.fori_loop(0, rp, clipmax, zval)
            stage8[0, :] = mx
            pltpu.sync_copy(stage8,
                            mx_tab.at[pl.ds(pl.multiple_of(s * 8, 8), 8)])
            plsc.subcore_barrier()
            pltpu.sync_copy(mx_tab, mxall)

            def red(j, m):
                return jnp.maximum(m, mxall[j * 8, :])

            gmx = lax.fori_loop(0, N_TILES, red, zval)
            gmx = jnp.where(gmx > 0.0, gmx, 1.0)
            ginv = 1.0 / gmx

            def scale(j, _):
                nbuf[j, :] = nbuf[j, :] * ginv
                return 0

            lax.fori_loop(0, rp, scale, 0)
            out_off = pl.multiple_of((c * NSLOT + 1 + t) * E + row0, 8)
            pltpu.sync_copy(nbuf, eall_h.at[pl.ds(out_off, rp)])
            if t < NUM_STEPS - 1:
                lax.fori_loop(0, rp, zero, 0)
                pltpu.sync_copy(nbuf, acc_tab.at[pl.ds(row0, rp)])
            plsc.subcore_barrier()

    mesh = plsc.VectorSubcoreMesh(core_axis_name="c", subcore_axis_name="s",
                                  num_cores=NUM_WAYS, num_subcores=N_TILES)
    return pl.kernel(
        body,
        out_type=jax.ShapeDtypeStruct((NUM_WAYS * NSLOT * E, 16),
                                      jnp.float32),
        mesh=mesh,
        compiler_params=pltpu.CompilerParams(use_tc_tiling_on_sc=False),
        scratch_types=[
            pltpu.VMEM_SHARED((E, 16), jnp.float32),
            pltpu.VMEM_SHARED((NUM_STEPS * R, 16), jnp.float32),
            pltpu.VMEM_SHARED((N_TILES * 8, 16), jnp.float32),
            pltpu.VMEM((CHUNK,), jnp.int32),
            pltpu.VMEM((NSUBS, SUBS), jnp.int32),
            pltpu.VMEM((CHUNK,), jnp.int32),
            pltpu.VMEM((CHUNK, 16), jnp.float32),
            pltpu.VMEM((CHUNK, 16), jnp.float32),
            pltpu.VMEM((E // N_TILES, 16), jnp.float32),
            pltpu.VMEM((8, 16), jnp.float32),
            pltpu.VMEM((1, 16), jnp.float32),
            pltpu.VMEM((N_TILES * 8, 16), jnp.float32),
            pltpu.SemaphoreType.DMA,
            pltpu.SemaphoreType.DMA,
            pltpu.SemaphoreType.DMA,
        ],
    )


def kernel(heads, q_embeddings, q_word_h, attention_mask, entity_emb,
           W_match, b_match, W_steps, b_steps, W_rel, b_rel, W_hop, b_hop,
           relation_importance, temperature,
           triples_subj, triples_rel, triples_obj):
    E, D = entity_emb.shape
    B = heads.shape[0]
    L = q_word_h.shape[1]
    R = W_rel.shape[2]
    T = triples_subj.shape[0]
    EB = 1000

    f32 = jnp.float32
    imp2 = relation_importance.reshape(1, R).astype(f32)
    temp2 = jnp.asarray(temperature, f32).reshape(1, 1)
    bm2 = b_match.reshape(1, D)

    qmT, wr16 = pl.pallas_call(
        _relmix_body,
        grid=(NUM_WAYS, NUM_STEPS),
        in_specs=[
            pl.BlockSpec((B, D), lambda w, t: (0, 0)),
            pl.BlockSpec((B, L, D), lambda w, t: (0, 0, 0)),
            pl.BlockSpec((B, L), lambda w, t: (0, 0)),
            pl.BlockSpec((D, D), lambda w, t: (0, 0)),
            pl.BlockSpec((1, D), lambda w, t: (0, 0)),
            pl.BlockSpec((1, 1, D, D), lambda w, t: (w, t, 0, 0)),
            pl.BlockSpec((1, 1, D), lambda w, t: (w * NUM_STEPS + t, 0, 0)),
            pl.BlockSpec((1, D, R), lambda w, t: (w, 0, 0)),
            pl.BlockSpec((1, 1, R), lambda w, t: (w, 0, 0)),
            pl.BlockSpec((1, R), lambda w, t: (0, 0)),
            pl.BlockSpec((1, 1), lambda w, t: (0, 0)),
        ],
        out_specs=[
            pl.BlockSpec((D, B), lambda w, t: (0, 0)),
            pl.BlockSpec((1, 1, R, 16), lambda w, t: (w, t, 0, 0)),
        ],
        out_shape=[
            jax.ShapeDtypeStruct((D, B), f32),
            jax.ShapeDtypeStruct((NUM_WAYS, NUM_STEPS, R, 16), f32),
        ],
    )(q_embeddings, q_word_h, attention_mask, W_match, bm2, W_steps,
      b_steps.reshape(NUM_WAYS * NUM_STEPS, 1, D),
      W_rel, b_rel.reshape(NUM_WAYS, 1, R), imp2, temp2)

    headsT = heads.T
    enh16, sums = pl.pallas_call(
        _direct_body,
        grid=(E // EB,),
        in_specs=[
            pl.BlockSpec((EB, D), lambda k: (k, 0)),
            pl.BlockSpec((D, B), lambda k: (0, 0)),
            pl.BlockSpec((EB, B), lambda k: (k, 0)),
        ],
        out_specs=[
            pl.BlockSpec((EB, 16), lambda k: (k, 0)),
            pl.BlockSpec((1, 16), lambda k: (0, 0)),
        ],
        out_shape=[
            jax.ShapeDtypeStruct((E, 16), f32),
            jax.ShapeDtypeStruct((1, 16), f32),
        ],
    )(entity_emb, qmT, headsT)

    EP = -(-E // (8 * N_TILES)) * (8 * N_TILES)
    tpt_p = -(-(T // N_TILES) // CHUNK) * CHUNK
    TP = tpt_p * N_TILES
    i32 = jnp.int32
    subjp = jnp.concatenate(
        [triples_subj, jnp.full((TP - T,), E, i32)])
    relp = jnp.concatenate([triples_rel, jnp.zeros((TP - T,), i32)])
    objp = jnp.concatenate([triples_obj, jnp.full((TP - T,), E, i32)])
    wr3 = wr16.reshape(NUM_WAYS, NUM_STEPS * R, 16)
    rel3 = (relp[None, :]
            + (jnp.arange(NUM_STEPS, dtype=i32) * R)[:, None]).reshape(-1)
    obj2 = objp.reshape(TP // SUBS, SUBS)
    enh16p = jnp.concatenate(
        [enh16, jnp.zeros((EP - E, 16), f32)], axis=0)
    eall = _make_sc_kernel(EP, TP, R)(enh16p, sums, wr3, subjp, rel3, obj2)

    probs3 = eall.reshape(NUM_WAYS, NSLOT, EP, 16)[:, 1:, :E].reshape(
        NUM_WAYS * NUM_STEPS, E, 16)
    outT = pl.pallas_call(
        _combine_body,
        grid=(E // EB,),
        in_specs=[
            pl.BlockSpec((NUM_WAYS * NUM_STEPS, EB, 16), lambda k: (0, k, 0)),
            pl.BlockSpec((B, D), lambda k: (0, 0)),
            pl.BlockSpec((NUM_WAYS, D, NUM_STEPS), lambda k: (0, 0, 0)),
            pl.BlockSpec((NUM_WAYS, NUM_STEPS), lambda k: (0, 0)),
        ],
        out_specs=pl.BlockSpec((EB, B), lambda k: (k, 0)),
        out_shape=jax.ShapeDtypeStruct((E, B), f32),
    )(probs3, q_embeddings, W_hop, b_hop)

    return outT.T


# speedup vs baseline: 14.7516x; 1.3486x over previous
import math

import jax
import jax.numpy as jnp
from jax import lax
from jax.experimental import pallas as pl
from jax.experimental.pallas import tpu as pltpu
from jax.experimental.pallas import tpu_sc as plsc

NUM_WAYS = 2
NUM_STEPS = 3
N_TILES = 16
CHUNK = 512
SUBG = 128
NSUBG = CHUNK // SUBG
SUBS = 64
NSUBS = CHUNK // SUBS
NSLOT = 4


def _relmix_body(q_ref, qwh_ref, mask_ref, Wm_ref, bm_ref, Wst_ref, bst_ref,
                 Wrel_ref, brel_ref, imp_ref, temp_ref, qmT_ref, wr16_ref):
    q = q_ref[...]
    qwh = qwh_ref[...]
    mask = mask_ref[...]
    temp = temp_ref[0, 0]
    qmT_ref[...] = (q @ Wm_ref[...] + bm_ref[...]).T
    W = Wst_ref[0, 0]
    b = bst_ref[0, 0]
    cq = jnp.tanh(q @ W + b)
    ql = jnp.sum(cq[:, None, :] * qwh, axis=2) / temp
    qd = jax.nn.softmax(ql, axis=1) * mask
    qd = qd / (jnp.sum(qd, axis=1, keepdims=True) + 1e-6)
    ctx = jnp.sum(qd[:, :, None] * qwh, axis=1)
    rl = ctx @ Wrel_ref[0] + brel_ref[0]
    rd = jax.nn.softmax(rl, axis=1)
    wr = rd * imp_ref[...]
    wrt = wr.T
    wr16_ref[0, 0] = jnp.concatenate([wrt, wrt], axis=1)


def _direct_body(emb_ref, qmT_ref, headsT_ref, enh16_ref, sums_ref):
    k = pl.program_id(0)
    d = emb_ref.shape[1]
    z = (emb_ref[...] @ qmT_ref[...]) * (1.0 / math.sqrt(d))
    dt = jax.nn.sigmoid(z)
    enh = headsT_ref[...] * (1.0 + 0.3 * dt)
    e16 = jnp.concatenate([enh, enh], axis=1)
    enh16_ref[...] = e16

    @pl.when(k == 0)
    def _():
        sums_ref[...] = jnp.zeros_like(sums_ref)

    sums_ref[...] += jnp.sum(e16, axis=0, keepdims=True)


def _combine_body(probs_ref, q_ref, Whop_ref, bhop_ref, outT_ref):
    q = q_ref[...]
    h0 = jax.nn.softmax(q @ Whop_ref[0] + bhop_ref[0], axis=1)
    h1 = jax.nn.softmax(q @ Whop_ref[1] + bhop_ref[1], axis=1)
    wgt = 0.5 * jnp.concatenate([h0.T, h1.T], axis=0)
    p8 = probs_ref[...][:, :, :8]
    outT_ref[...] = jnp.sum(p8 * wgt[:, None, :], axis=0)


def _make_sc_kernel(E, T, R):
    rp = E // N_TILES
    tpt = T // N_TILES
    nch = tpt // CHUNK

    def body(enh16_h, sums_h, wr3_h, subj_h, rel3_h, obj2_h, eall_h,
             acc_tab, wr_tab, mx_tab, idx_s, idx_o, idx_r, egbuf,
             rgbuf, nbuf, stage8, sumbuf, mxall, isem, gsem, ssem):
        c = lax.axis_index("c")
        s = lax.axis_index("s")
        row0 = pl.multiple_of(s * rp, 8)
        tri0 = pl.multiple_of(s * tpt, 8)
        zval = jnp.zeros((16,), jnp.float32)

        @pl.when(s == 0)
        def _():
            pltpu.sync_copy(wr3_h.at[c], wr_tab)

        pltpu.sync_copy(sums_h, sumbuf)
        hs = sumbuf[0, :]
        hs = jnp.where(hs > 0.0, hs, 1.0)
        hinv = 1.0 / hs
        pltpu.sync_copy(enh16_h.at[pl.ds(row0, rp)], nbuf)

        def scale0(j, _):
            nbuf[j, :] = nbuf[j, :] * hinv
            return 0

        lax.fori_loop(0, rp, scale0, 0)
        e0_off = pl.multiple_of(c * NSLOT * E + row0, 8)
        pltpu.sync_copy(nbuf, eall_h.at[pl.ds(e0_off, rp)])

        def zero(j, _):
            nbuf[j, :] = zval
            return 0

        lax.fori_loop(0, rp, zero, 0)
        pltpu.sync_copy(nbuf, acc_tab.at[pl.ds(row0, rp)])
        plsc.subcore_barrier()

        nvec = CHUNK // 16
        for t in range(NUM_STEPS):

            def chunk(g, _):
                t0 = pl.multiple_of(tri0 + g * CHUNK, 8)
                di = [
                    pltpu.async_copy(subj_h.at[pl.ds(t0, CHUNK)], idx_s,
                                     isem),
                    pltpu.async_copy(rel3_h.at[pl.ds(t * T + t0, CHUNK)],
                                     idx_r, isem),
                    pltpu.async_copy(
                        obj2_h.at[pl.ds(pl.multiple_of(t0 // SUBS, 8),
                                        NSUBS)], idx_o, isem),
                ]
                for d in di:
                    d.wait()
                src_base = (c * NSLOT + t) * E

                @plsc.parallel_loop(0, nvec, unroll=4)
                def shift(i):
                    sl = pl.ds(pl.multiple_of(i * 16, 8), 16)
                    idx_s[sl] = idx_s[sl] + src_base
                dg = []
                for j in range(NSUBG):
                    dg.append(pltpu.async_copy(
                        eall_h.at[idx_s.at[pl.ds(j * SUBG, SUBG)]],
                        egbuf.at[pl.ds(j * SUBG, SUBG)], gsem))
                for j in range(NSUBG):
                    pltpu.sync_copy(wr_tab.at[idx_r.at[pl.ds(j * SUBG, SUBG)]],
                                    rgbuf.at[pl.ds(j * SUBG, SUBG)])
                for d in dg:
                    d.wait()

                @plsc.parallel_loop(0, CHUNK, unroll=8)
                def mul(m):
                    egbuf[m, :] = egbuf[m, :] * rgbuf[m, :]
                ds_ = []
                for j in range(NSUBS):
                    ds_.append(pltpu.async_copy(
                        egbuf.at[pl.ds(j * SUBS, SUBS)],
                        acc_tab.at[idx_o.at[j]], ssem, add=True))
                for d in ds_:
                    d.wait()
                return 0

            lax.fori_loop(0, nch, chunk, 0)
            plsc.subcore_barrier()

            pltpu.sync_copy(acc_tab.at[pl.ds(row0, rp)], nbuf)

            def clipmax(j, mx):
                v = nbuf[j, :]
                v = jnp.minimum(jnp.maximum(v, 0.0), 1.0)
                nbuf[j, :] = v
                return jnp.maximum(mx, v)

            mx = lax.fori_loop(0, rp, clipmax, zval)
            stage8[0, :] = mx
            pltpu.sync_copy(stage8,
                            mx_tab.at[pl.ds(pl.multiple_of(s * 8, 8), 8)])
            plsc.subcore_barrier()
            pltpu.sync_copy(mx_tab, mxall)

            def red(j, m):
                return jnp.maximum(m, mxall[j * 8, :])

            gmx = lax.fori_loop(0, N_TILES, red, zval)
            gmx = jnp.where(gmx > 0.0, gmx, 1.0)
            ginv = 1.0 / gmx

            def scale(j, _):
                nbuf[j, :] = nbuf[j, :] * ginv
                return 0

            lax.fori_loop(0, rp, scale, 0)
            out_off = pl.multiple_of((c * NSLOT + 1 + t) * E + row0, 8)
            pltpu.sync_copy(nbuf, eall_h.at[pl.ds(out_off, rp)])
            if t < NUM_STEPS - 1:
                lax.fori_loop(0, rp, zero, 0)
                pltpu.sync_copy(nbuf, acc_tab.at[pl.ds(row0, rp)])
            plsc.subcore_barrier()

    mesh = plsc.VectorSubcoreMesh(core_axis_name="c", subcore_axis_name="s",
                                  num_cores=NUM_WAYS, num_subcores=N_TILES)
    return pl.kernel(
        body,
        out_type=jax.ShapeDtypeStruct((NUM_WAYS * NSLOT * E, 16),
                                      jnp.float32),
        mesh=mesh,
        compiler_params=pltpu.CompilerParams(use_tc_tiling_on_sc=False),
        scratch_types=[
            pltpu.VMEM_SHARED((E, 16), jnp.float32),
            pltpu.VMEM_SHARED((NUM_STEPS * R, 16), jnp.float32),
            pltpu.VMEM_SHARED((N_TILES * 8, 16), jnp.float32),
            pltpu.VMEM((CHUNK,), jnp.int32),
            pltpu.VMEM((NSUBS, SUBS), jnp.int32),
            pltpu.VMEM((CHUNK,), jnp.int32),
            pltpu.VMEM((CHUNK, 16), jnp.float32),
            pltpu.VMEM((CHUNK, 16), jnp.float32),
            pltpu.VMEM((E // N_TILES, 16), jnp.float32),
            pltpu.VMEM((8, 16), jnp.float32),
            pltpu.VMEM((1, 16), jnp.float32),
            pltpu.VMEM((N_TILES * 8, 16), jnp.float32),
            pltpu.SemaphoreType.DMA,
            pltpu.SemaphoreType.DMA,
            pltpu.SemaphoreType.DMA,
        ],
    )


def kernel(heads, q_embeddings, q_word_h, attention_mask, entity_emb,
           W_match, b_match, W_steps, b_steps, W_rel, b_rel, W_hop, b_hop,
           relation_importance, temperature,
           triples_subj, triples_rel, triples_obj):
    E, D = entity_emb.shape
    B = heads.shape[0]
    L = q_word_h.shape[1]
    R = W_rel.shape[2]
    T = triples_subj.shape[0]
    EB = 1000

    f32 = jnp.float32
    imp2 = relation_importance.reshape(1, R).astype(f32)
    temp2 = jnp.asarray(temperature, f32).reshape(1, 1)
    bm2 = b_match.reshape(1, D)

    qmT, wr16 = pl.pallas_call(
        _relmix_body,
        grid=(NUM_WAYS, NUM_STEPS),
        in_specs=[
            pl.BlockSpec((B, D), lambda w, t: (0, 0)),
            pl.BlockSpec((B, L, D), lambda w, t: (0, 0, 0)),
            pl.BlockSpec((B, L), lambda w, t: (0, 0)),
            pl.BlockSpec((D, D), lambda w, t: (0, 0)),
            pl.BlockSpec((1, D), lambda w, t: (0, 0)),
            pl.BlockSpec((1, 1, D, D), lambda w, t: (w, t, 0, 0)),
            pl.BlockSpec((1, 1, D), lambda w, t: (w * NUM_STEPS + t, 0, 0)),
            pl.BlockSpec((1, D, R), lambda w, t: (w, 0, 0)),
            pl.BlockSpec((1, 1, R), lambda w, t: (w, 0, 0)),
            pl.BlockSpec((1, R), lambda w, t: (0, 0)),
            pl.BlockSpec((1, 1), lambda w, t: (0, 0)),
        ],
        out_specs=[
            pl.BlockSpec((D, B), lambda w, t: (0, 0)),
            pl.BlockSpec((1, 1, R, 16), lambda w, t: (w, t, 0, 0)),
        ],
        out_shape=[
            jax.ShapeDtypeStruct((D, B), f32),
            jax.ShapeDtypeStruct((NUM_WAYS, NUM_STEPS, R, 16), f32),
        ],
    )(q_embeddings, q_word_h, attention_mask, W_match, bm2, W_steps,
      b_steps.reshape(NUM_WAYS * NUM_STEPS, 1, D),
      W_rel, b_rel.reshape(NUM_WAYS, 1, R), imp2, temp2)

    headsT = heads.T
    enh16, sums = pl.pallas_call(
        _direct_body,
        grid=(E // EB,),
        in_specs=[
            pl.BlockSpec((EB, D), lambda k: (k, 0)),
            pl.BlockSpec((D, B), lambda k: (0, 0)),
            pl.BlockSpec((EB, B), lambda k: (k, 0)),
        ],
        out_specs=[
            pl.BlockSpec((EB, 16), lambda k: (k, 0)),
            pl.BlockSpec((1, 16), lambda k: (0, 0)),
        ],
        out_shape=[
            jax.ShapeDtypeStruct((E, 16), f32),
            jax.ShapeDtypeStruct((1, 16), f32),
        ],
    )(entity_emb, qmT, headsT)

    EP = -(-E // (8 * N_TILES)) * (8 * N_TILES)
    tpt_p = -(-(T // N_TILES) // CHUNK) * CHUNK
    TP = tpt_p * N_TILES
    i32 = jnp.int32
    subjp = jnp.concatenate(
        [triples_subj, jnp.full((TP - T,), E, i32)])
    relp = jnp.concatenate([triples_rel, jnp.zeros((TP - T,), i32)])
    objp = jnp.concatenate([triples_obj, jnp.full((TP - T,), E, i32)])
    wr3 = wr16.reshape(NUM_WAYS, NUM_STEPS * R, 16)
    rel3 = (relp[None, :]
            + (jnp.arange(NUM_STEPS, dtype=i32) * R)[:, None]).reshape(-1)
    obj2 = objp.reshape(TP // SUBS, SUBS)
    enh16p = jnp.concatenate(
        [enh16, jnp.zeros((EP - E, 16), f32)], axis=0)
    eall = _make_sc_kernel(EP, TP, R)(enh16p, sums, wr3, subjp, rel3, obj2)

    probs3 = eall.reshape(NUM_WAYS, NSLOT, EP, 16)[:, 1:, :E].reshape(
        NUM_WAYS * NUM_STEPS, E, 16)
    outT = pl.pallas_call(
        _combine_body,
        grid=(E // EB,),
        in_specs=[
            pl.BlockSpec((NUM_WAYS * NUM_STEPS, EB, 16), lambda k: (0, k, 0)),
            pl.BlockSpec((B, D), lambda k: (0, 0)),
            pl.BlockSpec((NUM_WAYS, D, NUM_STEPS), lambda k: (0, 0, 0)),
            pl.BlockSpec((NUM_WAYS, NUM_STEPS), lambda k: (0, 0)),
        ],
        out_specs=pl.BlockSpec((EB, B), lambda k: (k, 0)),
        out_shape=jax.ShapeDtypeStruct((E, B), f32),
    )(probs3, q_embeddings, W_hop, b_hop)

    return outT.T


# speedup vs baseline: 16.7553x; 1.1358x over previous
import math

import jax
import jax.numpy as jnp
from jax import lax
from jax.experimental import pallas as pl
from jax.experimental.pallas import tpu as pltpu
from jax.experimental.pallas import tpu_sc as plsc

NUM_WAYS = 2
NUM_STEPS = 3
N_TILES = 16
CHUNK = 512
SUBG = 128
NSUBG = CHUNK // SUBG
SUBS = 64
NSUBS = CHUNK // SUBS
NSLOT = 4


def _relmix_body(q_ref, qwh_ref, mask_ref, Wm_ref, bm_ref, Wst_ref, bst_ref,
                 Wrel_ref, brel_ref, imp_ref, temp_ref, qmT_ref, wr16_ref):
    q = q_ref[...]
    qwh = qwh_ref[...]
    mask = mask_ref[...]
    temp = temp_ref[0, 0]
    qmT_ref[...] = (q @ Wm_ref[...] + bm_ref[...]).T
    W = Wst_ref[0, 0]
    b = bst_ref[0, 0]
    cq = jnp.tanh(q @ W + b)
    ql = jnp.sum(cq[:, None, :] * qwh, axis=2) / temp
    qd = jax.nn.softmax(ql, axis=1) * mask
    qd = qd / (jnp.sum(qd, axis=1, keepdims=True) + 1e-6)
    ctx = jnp.sum(qd[:, :, None] * qwh, axis=1)
    rl = ctx @ Wrel_ref[0] + brel_ref[0]
    rd = jax.nn.softmax(rl, axis=1)
    wr = rd * imp_ref[...]
    wrt = wr.T
    wr16_ref[0, 0] = jnp.concatenate([wrt, wrt], axis=1)


def _direct_body(emb_ref, qmT_ref, headsT_ref, enh16_ref, sums_ref):
    k = pl.program_id(0)
    d = emb_ref.shape[1]
    z = (emb_ref[...] @ qmT_ref[...]) * (1.0 / math.sqrt(d))
    dt = jax.nn.sigmoid(z)
    enh = headsT_ref[...] * (1.0 + 0.3 * dt)
    e16 = jnp.concatenate([enh, enh], axis=1)
    enh16_ref[...] = e16

    @pl.when(k == 0)
    def _():
        sums_ref[...] = jnp.zeros_like(sums_ref)

    sums_ref[...] += jnp.sum(e16, axis=0, keepdims=True)


def _combine_body(probs_ref, q_ref, Whop_ref, bhop_ref, outT_ref):
    q = q_ref[...]
    h0 = jax.nn.softmax(q @ Whop_ref[0] + bhop_ref[0], axis=1)
    h1 = jax.nn.softmax(q @ Whop_ref[1] + bhop_ref[1], axis=1)
    wgt = 0.5 * jnp.concatenate([h0.T, h1.T], axis=0)
    p = probs_ref[...]
    p8 = jnp.concatenate([p[1:4], p[5:8]], axis=0)[:, :, :8]
    outT_ref[...] = jnp.sum(p8 * wgt[:, None, :], axis=0)


def _make_sc_kernel(E, T, R):
    rp = E // N_TILES
    tpt = T // N_TILES
    nch = tpt // CHUNK

    def body(enh16_h, sums_h, wr3_h, subj_h, rel3_h, obj2_h, eall_h,
             acc_tab, wr_tab, mx_tab, idx_s, idx_o, idx_r, egbuf,
             rgbuf, nbuf, stage8, sumbuf, mxall, isem, gsem, ssem):
        c = lax.axis_index("c")
        s = lax.axis_index("s")
        row0 = pl.multiple_of(s * rp, 8)
        tri0 = pl.multiple_of(s * tpt, 8)
        zval = jnp.zeros((16,), jnp.float32)

        @pl.when(s == 0)
        def _():
            pltpu.sync_copy(wr3_h.at[c], wr_tab)

        pltpu.sync_copy(sums_h, sumbuf)
        hs = sumbuf[0, :]
        hs = jnp.where(hs > 0.0, hs, 1.0)
        hinv = 1.0 / hs
        pltpu.sync_copy(enh16_h.at[pl.ds(row0, rp)], nbuf)

        @plsc.parallel_loop(0, rp, unroll=8)
        def scale0(j):
            nbuf[j, :] = nbuf[j, :] * hinv
        e0_off = pl.multiple_of(c * NSLOT * E + row0, 8)
        pltpu.sync_copy(nbuf, eall_h.at[pl.ds(e0_off, rp)])

        def zero_nbuf():
            @plsc.parallel_loop(0, rp, unroll=8)
            def zero(j):
                nbuf[j, :] = zval

        zero_nbuf()
        pltpu.sync_copy(nbuf, acc_tab.at[pl.ds(row0, rp)])
        plsc.subcore_barrier()

        nvec = CHUNK // 16
        for t in range(NUM_STEPS):

            def chunk(g, _):
                t0 = pl.multiple_of(tri0 + g * CHUNK, 8)
                di = [
                    pltpu.async_copy(subj_h.at[pl.ds(t0, CHUNK)], idx_s,
                                     isem),
                    pltpu.async_copy(rel3_h.at[pl.ds(t * T + t0, CHUNK)],
                                     idx_r, isem),
                    pltpu.async_copy(
                        obj2_h.at[pl.ds(pl.multiple_of(t0 // SUBS, 8),
                                        NSUBS)], idx_o, isem),
                ]
                for d in di:
                    d.wait()
                src_base = (c * NSLOT + t) * E

                @plsc.parallel_loop(0, nvec, unroll=4)
                def shift(i):
                    sl = pl.ds(pl.multiple_of(i * 16, 8), 16)
                    idx_s[sl] = idx_s[sl] + src_base
                dg = []
                for j in range(NSUBG):
                    dg.append(pltpu.async_copy(
                        eall_h.at[idx_s.at[pl.ds(j * SUBG, SUBG)]],
                        egbuf.at[pl.ds(j * SUBG, SUBG)], gsem))
                for j in range(NSUBG):
                    pltpu.sync_copy(wr_tab.at[idx_r.at[pl.ds(j * SUBG, SUBG)]],
                                    rgbuf.at[pl.ds(j * SUBG, SUBG)])
                for d in dg:
                    d.wait()

                @plsc.parallel_loop(0, CHUNK, unroll=8)
                def mul(m):
                    egbuf[m, :] = egbuf[m, :] * rgbuf[m, :]
                ds_ = []
                for j in range(NSUBS):
                    ds_.append(pltpu.async_copy(
                        egbuf.at[pl.ds(j * SUBS, SUBS)],
                        acc_tab.at[idx_o.at[j]], ssem, add=True))
                for d in ds_:
                    d.wait()
                return 0

            lax.fori_loop(0, nch, chunk, 0)
            plsc.subcore_barrier()

            pltpu.sync_copy(acc_tab.at[pl.ds(row0, rp)], nbuf)

            @plsc.parallel_loop(0, rp, unroll=8, carry=zval)
            def clipmax(j, mx):
                v = nbuf[j, :]
                v = jnp.minimum(jnp.maximum(v, 0.0), 1.0)
                nbuf[j, :] = v
                return jnp.maximum(mx, v)

            mx = clipmax
            stage8[0, :] = mx
            pltpu.sync_copy(stage8,
                            mx_tab.at[pl.ds(pl.multiple_of(s * 8, 8), 8)])
            plsc.subcore_barrier()
            pltpu.sync_copy(mx_tab, mxall)

            def red(j, m):
                return jnp.maximum(m, mxall[j * 8, :])

            gmx = lax.fori_loop(0, N_TILES, red, zval)
            gmx = jnp.where(gmx > 0.0, gmx, 1.0)
            ginv = 1.0 / gmx

            @plsc.parallel_loop(0, rp, unroll=8)
            def scale(j):
                nbuf[j, :] = nbuf[j, :] * ginv

            out_off = pl.multiple_of((c * NSLOT + 1 + t) * E + row0, 8)
            pltpu.sync_copy(nbuf, eall_h.at[pl.ds(out_off, rp)])
            if t < NUM_STEPS - 1:
                zero_nbuf()
                pltpu.sync_copy(nbuf, acc_tab.at[pl.ds(row0, rp)])
            plsc.subcore_barrier()

    mesh = plsc.VectorSubcoreMesh(core_axis_name="c", subcore_axis_name="s",
                                  num_cores=NUM_WAYS, num_subcores=N_TILES)
    return pl.kernel(
        body,
        out_type=jax.ShapeDtypeStruct((NUM_WAYS * NSLOT * E, 16),
                                      jnp.float32),
        mesh=mesh,
        compiler_params=pltpu.CompilerParams(use_tc_tiling_on_sc=False),
        scratch_types=[
            pltpu.VMEM_SHARED((E, 16), jnp.float32),
            pltpu.VMEM_SHARED((NUM_STEPS * R, 16), jnp.float32),
            pltpu.VMEM_SHARED((N_TILES * 8, 16), jnp.float32),
            pltpu.VMEM((CHUNK,), jnp.int32),
            pltpu.VMEM((NSUBS, SUBS), jnp.int32),
            pltpu.VMEM((CHUNK,), jnp.int32),
            pltpu.VMEM((CHUNK, 16), jnp.float32),
            pltpu.VMEM((CHUNK, 16), jnp.float32),
            pltpu.VMEM((E // N_TILES, 16), jnp.float32),
            pltpu.VMEM((8, 16), jnp.float32),
            pltpu.VMEM((1, 16), jnp.float32),
            pltpu.VMEM((N_TILES * 8, 16), jnp.float32),
            pltpu.SemaphoreType.DMA,
            pltpu.SemaphoreType.DMA,
            pltpu.SemaphoreType.DMA,
        ],
    )


def kernel(heads, q_embeddings, q_word_h, attention_mask, entity_emb,
           W_match, b_match, W_steps, b_steps, W_rel, b_rel, W_hop, b_hop,
           relation_importance, temperature,
           triples_subj, triples_rel, triples_obj):
    E, D = entity_emb.shape
    B = heads.shape[0]
    L = q_word_h.shape[1]
    R = W_rel.shape[2]
    T = triples_subj.shape[0]
    EB = 1000

    f32 = jnp.float32
    imp2 = relation_importance.reshape(1, R).astype(f32)
    temp2 = jnp.asarray(temperature, f32).reshape(1, 1)
    bm2 = b_match.reshape(1, D)

    qmT, wr16 = pl.pallas_call(
        _relmix_body,
        grid=(NUM_WAYS, NUM_STEPS),
        in_specs=[
            pl.BlockSpec((B, D), lambda w, t: (0, 0)),
            pl.BlockSpec((B, L, D), lambda w, t: (0, 0, 0)),
            pl.BlockSpec((B, L), lambda w, t: (0, 0)),
            pl.BlockSpec((D, D), lambda w, t: (0, 0)),
            pl.BlockSpec((1, D), lambda w, t: (0, 0)),
            pl.BlockSpec((1, 1, D, D), lambda w, t: (w, t, 0, 0)),
            pl.BlockSpec((1, 1, D), lambda w, t: (w * NUM_STEPS + t, 0, 0)),
            pl.BlockSpec((1, D, R), lambda w, t: (w, 0, 0)),
            pl.BlockSpec((1, 1, R), lambda w, t: (w, 0, 0)),
            pl.BlockSpec((1, R), lambda w, t: (0, 0)),
            pl.BlockSpec((1, 1), lambda w, t: (0, 0)),
        ],
        out_specs=[
            pl.BlockSpec((D, B), lambda w, t: (0, 0)),
            pl.BlockSpec((1, 1, R, 16), lambda w, t: (w, t, 0, 0)),
        ],
        out_shape=[
            jax.ShapeDtypeStruct((D, B), f32),
            jax.ShapeDtypeStruct((NUM_WAYS, NUM_STEPS, R, 16), f32),
        ],
    )(q_embeddings, q_word_h, attention_mask, W_match, bm2, W_steps,
      b_steps.reshape(NUM_WAYS * NUM_STEPS, 1, D),
      W_rel, b_rel.reshape(NUM_WAYS, 1, R), imp2, temp2)

    headsT = heads.T
    enh16, sums = pl.pallas_call(
        _direct_body,
        grid=(E // EB,),
        in_specs=[
            pl.BlockSpec((EB, D), lambda k: (k, 0)),
            pl.BlockSpec((D, B), lambda k: (0, 0)),
            pl.BlockSpec((EB, B), lambda k: (k, 0)),
        ],
        out_specs=[
            pl.BlockSpec((EB, 16), lambda k: (k, 0)),
            pl.BlockSpec((1, 16), lambda k: (0, 0)),
        ],
        out_shape=[
            jax.ShapeDtypeStruct((E, 16), f32),
            jax.ShapeDtypeStruct((1, 16), f32),
        ],
    )(entity_emb, qmT, headsT)

    EP = -(-E // (8 * N_TILES)) * (8 * N_TILES)
    tpt_p = -(-(T // N_TILES) // CHUNK) * CHUNK
    TP = tpt_p * N_TILES
    i32 = jnp.int32
    subjp = jnp.concatenate(
        [triples_subj, jnp.full((TP - T,), E, i32)])
    relp = jnp.concatenate([triples_rel, jnp.zeros((TP - T,), i32)])
    objp = jnp.concatenate([triples_obj, jnp.full((TP - T,), E, i32)])
    wr3 = wr16.reshape(NUM_WAYS, NUM_STEPS * R, 16)
    rel3 = (relp[None, :]
            + (jnp.arange(NUM_STEPS, dtype=i32) * R)[:, None]).reshape(-1)
    obj2 = objp.reshape(TP // SUBS, SUBS)
    enh16p = jnp.concatenate(
        [enh16, jnp.zeros((EP - E, 16), f32)], axis=0)
    eall = _make_sc_kernel(EP, TP, R)(enh16p, sums, wr3, subjp, rel3, obj2)

    probs3 = eall.reshape(NUM_WAYS * NSLOT, EP, 16)
    outT = pl.pallas_call(
        _combine_body,
        grid=(E // EB,),
        in_specs=[
            pl.BlockSpec((NUM_WAYS * NSLOT, EB, 16), lambda k: (0, k, 0)),
            pl.BlockSpec((B, D), lambda k: (0, 0)),
            pl.BlockSpec((NUM_WAYS, D, NUM_STEPS), lambda k: (0, 0, 0)),
            pl.BlockSpec((NUM_WAYS, NUM_STEPS), lambda k: (0, 0)),
        ],
        out_specs=pl.BlockSpec((EB, B), lambda k: (k, 0)),
        out_shape=jax.ShapeDtypeStruct((E, B), f32),
    )(probs3, q_embeddings, W_hop, b_hop)

    return outT.T
